# baseline (speedup 1.0000x reference)
import jax
import jax.numpy as jnp
from jax import lax
from jax.experimental import pallas as pl
from jax.experimental.pallas import tpu as pltpu
from jax.experimental.pallas import tpu_sc as plsc

_D = 64
_L = 16
_NC = 2
_NS = 16
_NW = _NC * _NS
_C = 128
_NBUF = 4
_SCALE = 8.0


def _make_sc_gather(nchunk: int):
  mesh = plsc.VectorSubcoreMesh(core_axis_name="c", subcore_axis_name="s")
  b_per_w = nchunk * _C
  n_rows = b_per_w * _NW

  def body(idx_hbm, table_hbm, out_hbm, idx_v, gbuf, obuf, *sems):
    gsems = sems[:_NBUF]
    osems = sems[_NBUF:]
    wid = lax.axis_index("s") * _NC + lax.axis_index("c")
    base = wid * b_per_w

    pltpu.sync_copy(idx_hbm.at[wid], idx_v)

    def g_copy(j, b):
      return pltpu.make_async_copy(
          table_hbm.at[idx_v.at[j]], gbuf.at[b], gsems[b])

    def o_copy(j, b):
      return pltpu.make_async_copy(
          obuf.at[b], out_hbm.at[pl.ds(base + j * _C, _C)], osems[b])

    for b in range(_NBUF):
      g_copy(b, b).start()

    def outer(io, carry):
      jo = io * _NBUF
      for b in range(_NBUF):
        j = jo + b
        g_copy(j, b).wait()

        @pl.when(j >= _NBUF)
        def _():
          o_copy(j - _NBUF, b).wait()

        def srow(i, c):
          for l in range(_D // _L):
            s = pl.ds(l * _L, _L)
            obuf[b, i, s] = gbuf[b, i, s] * _SCALE
          return c
        lax.fori_loop(0, _C, srow, 0)

        @pl.when(j + _NBUF < nchunk)
        def _():
          g_copy(j + _NBUF, b).start()

        o_copy(j, b).start()
      return carry

    lax.fori_loop(0, nchunk // _NBUF, outer, 0)

    for b in range(_NBUF):
      o_copy(nchunk - _NBUF + b, b).wait()

  return pl.kernel(
      body,
      mesh=mesh,
      out_type=jax.ShapeDtypeStruct((n_rows, _D), jnp.float32),
      scratch_types=[
          pltpu.VMEM((nchunk, _C), jnp.int32),
          pltpu.VMEM((_NBUF, _C, _D), jnp.float32),
          pltpu.VMEM((_NBUF, _C, _D), jnp.float32),
      ] + [pltpu.SemaphoreType.DMA] * (2 * _NBUF),
      compiler_params=pltpu.CompilerParams(use_tc_tiling_on_sc=False),
  )


def kernel(x, table):
  n = x.size
  nchunk = n // (_NW * _C)
  assert n == nchunk * _NW * _C and nchunk % _NBUF == 0
  idx = x.reshape(_NW, nchunk, _C).astype(jnp.int32)
  out = _make_sc_gather(nchunk)(idx, table)
  return out.reshape(x.shape + (_D,))
